# baseline (speedup 1.0000x reference)
import functools

import jax
import jax.numpy as jnp
from jax import lax
from jax.experimental import pallas as pl
from jax.experimental.pallas import tpu as pltpu
from jax.experimental.pallas import tpu_sc as plsc

NUM_CODES = 8192
CODE_DIM = 256
N_TOK = 32 * 24 * 24
TILE = 256
N_TILES = N_TOK // TILE
CNT_ROWS = 1
BETA = 0.25
EPS = 1e-05

SC_NC = 2
SC_NS = 16
SC_NW = SC_NC * SC_NS
TOK_PER_W = N_TOK // SC_NW
CHUNK = 96
CHUNKS_PER_W = TOK_PER_W // CHUNK


def _assign_body(z_ref, zsq_ref, emb_ref, esq_ref, idx_ref, losssum_ref,
                 cnt_ref):
    i = pl.program_id(0)
    z = z_ref[...]
    dot = lax.dot_general(z, emb_ref[...], (((1,), (1,)), ((), ())),
                          preferred_element_type=jnp.float32)
    dists = zsq_ref[...] - 2.0 * dot + esq_ref[...]
    m = jnp.min(dists, axis=1)
    s = jnp.sum(m)
    oh = dists == m[:, None]
    ohb = oh.astype(jnp.bfloat16)
    ones_r = jnp.ones((CNT_ROWS, TILE), jnp.bfloat16)
    oh8 = lax.dot_general(ones_r, ohb, (((1,), (0,)), ((), ())),
                          preferred_element_type=jnp.float32)
    iota_c = lax.broadcasted_iota(jnp.int32, (NUM_CODES,), 0)
    digits = jnp.stack(
        [iota_c // 64, iota_c % 64, jnp.ones((NUM_CODES,), jnp.int32)],
        axis=1).astype(jnp.bfloat16)
    hls = lax.dot_general(ohb, digits, (((1,), (0,)), ((), ())),
                          preferred_element_type=jnp.float32)
    tc = hls[:, 2]
    idx_fast = (hls[:, 0] * 64.0 + hls[:, 1]).astype(jnp.int32)
    anytie = jnp.max(tc) > 1.5

    @pl.when(jnp.logical_not(anytie))
    def _fast():
        idx_ref[0, 0, :] = idx_fast

    @pl.when(anytie)
    def _slow():
        iota_k = lax.broadcasted_iota(jnp.int32, (TILE, NUM_CODES), 1)
        idx_ref[0, 0, :] = jnp.min(jnp.where(oh, iota_k, NUM_CODES), axis=1)

    @pl.when(i == 0)
    def _init():
        losssum_ref[0, 0] = s
        cnt_ref[...] = oh8

    @pl.when(i != 0)
    def _acc():
        losssum_ref[0, 0] += s
        cnt_ref[...] += oh8


_assign = pl.pallas_call(
    _assign_body,
    grid=(N_TILES,),
    in_specs=[
        pl.BlockSpec((TILE, CODE_DIM), lambda i: (i, 0)),
        pl.BlockSpec((TILE, 1), lambda i: (i, 0)),
        pl.BlockSpec((NUM_CODES, CODE_DIM), lambda i: (0, 0)),
        pl.BlockSpec((1, NUM_CODES), lambda i: (0, 0)),
    ],
    out_specs=[
        pl.BlockSpec((1, 1, TILE), lambda i: (i, 0, 0)),
        pl.BlockSpec((1, 1), lambda i: (0, 0), memory_space=pltpu.SMEM),
        pl.BlockSpec((CNT_ROWS, NUM_CODES), lambda i: (0, 0)),
    ],
    out_shape=[
        jax.ShapeDtypeStruct((N_TILES, 1, TILE), jnp.int32),
        jax.ShapeDtypeStruct((1, 1), jnp.float32),
        jax.ShapeDtypeStruct((CNT_ROWS, NUM_CODES), jnp.float32),
    ],
)


@functools.cache
def _make_sc_gather():
    return functools.partial(
        pl.kernel,
        mesh=plsc.VectorSubcoreMesh(core_axis_name="c", subcore_axis_name="s"),
        out_type=jax.ShapeDtypeStruct((N_TOK, CODE_DIM), jnp.float32),
        scratch_types=[
            pltpu.VMEM((CHUNKS_PER_W, CHUNK), jnp.int32),
            pltpu.VMEM((CHUNK, CODE_DIM), jnp.float32),
            pltpu.VMEM((CHUNK, CODE_DIM), jnp.float32),
            pltpu.SemaphoreType.DMA,
            pltpu.SemaphoreType.DMA,
        ],
    )(_sc_gather_body)


def _sc_gather_body(emb_hbm, idx_hbm, zq_hbm, idx_v, rows_a, rows_b, sem_a,
                    sem_b):
    c = lax.axis_index("c")
    s = lax.axis_index("s")
    wid = s * SC_NC + c
    pltpu.sync_copy(idx_hbm.at[wid], idx_v)

    bufs = (rows_a, rows_b)
    sems = (sem_a, sem_b)
    pltpu.async_copy(emb_hbm.at[idx_v.at[0]], rows_a, sem_a)
    for t in range(CHUNKS_PER_W):
        if t + 1 < CHUNKS_PER_W:
            pltpu.async_copy(emb_hbm.at[idx_v.at[t + 1]],
                             bufs[(t + 1) % 2], sems[(t + 1) % 2])
        pltpu.make_async_copy(emb_hbm.at[idx_v.at[t]], bufs[t % 2],
                              sems[t % 2]).wait()
        pltpu.sync_copy(bufs[t % 2],
                        zq_hbm.at[pl.ds(wid * TOK_PER_W + t * CHUNK, CHUNK)])


def _final_body(losssum_ref, cnt_ref, vq_ref, com_ref, cb_ref, perp_ref,
                use_ref):
    counts = jnp.sum(cnt_ref[...], axis=0)
    probs = jnp.maximum(counts / float(N_TOK), EPS)
    entropy = -jnp.sum(probs * jnp.log(probs))
    perp_ref[0, 0] = jnp.exp(entropy)
    use_ref[0, 0] = jnp.sum((counts > 0).astype(jnp.float32)) * (1.0 / NUM_CODES)
    ms = losssum_ref[0, 0] / float(N_TOK * CODE_DIM)
    com = jnp.minimum(BETA * ms, 100.0)
    cb = jnp.minimum(ms, 100.0)
    com_ref[0, 0] = com
    cb_ref[0, 0] = cb
    vq_ref[0, 0] = com + cb


_final = pl.pallas_call(
    _final_body,
    in_specs=[
        pl.BlockSpec((1, 1), lambda: (0, 0), memory_space=pltpu.SMEM),
        pl.BlockSpec((CNT_ROWS, NUM_CODES), lambda: (0, 0)),
    ],
    out_specs=[pl.BlockSpec((1, 1), lambda: (0, 0),
                            memory_space=pltpu.SMEM)] * 5,
    out_shape=[jax.ShapeDtypeStruct((1, 1), jnp.float32)] * 5,
)


def kernel(z_e, embedding):
    B, D, H, W = z_e.shape
    z = jnp.transpose(z_e, (0, 2, 3, 1))
    z_flat = z.reshape(-1, D)
    z_sq = jnp.sum(z_flat ** 2, axis=1, keepdims=True)
    e_sq = jnp.sum(embedding ** 2, axis=1)[None, :]
    idx3, losssum, cnt = _assign(z_flat, z_sq, embedding, e_sq)
    idx_flat = idx3.reshape(-1)
    zq_flat = _make_sc_gather()(
        embedding, idx_flat.reshape(SC_NW, CHUNKS_PER_W, CHUNK))
    vq, com, cb, perp, use = _final(losssum, cnt)
    z_q = jnp.transpose(zq_flat.reshape(B, H, W, D), (0, 3, 1, 2))
    idx_bt = idx_flat.reshape(B, H * W)[:, None, :]
    return (z_q, vq[0, 0], com[0, 0], cb[0, 0], perp[0, 0], use[0, 0], idx_bt)


# speedup vs baseline: 1.2655x; 1.2655x over previous
import functools

import jax
import jax.numpy as jnp
from jax import lax
from jax.experimental import pallas as pl
from jax.experimental.pallas import tpu as pltpu
from jax.experimental.pallas import tpu_sc as plsc

NUM_CODES = 8192
CODE_DIM = 256
N_TOK = 32 * 24 * 24
TILE = 512
N_TILES = N_TOK // TILE
CNT_ROWS = 1
BETA = 0.25
EPS = 1e-05

SC_NC = 2
SC_NS = 16
SC_NW = SC_NC * SC_NS
TOK_PER_W = N_TOK // SC_NW
CHUNK = 96
CHUNKS_PER_W = TOK_PER_W // CHUNK


def _assign_body(z_ref, zsq_ref, emb_ref, esq_ref, idx_ref, losssum_ref,
                 cnt_ref):
    i = pl.program_id(0)
    z = z_ref[...]
    dot = lax.dot_general(z, emb_ref[...], (((1,), (1,)), ((), ())),
                          preferred_element_type=jnp.float32)
    dists = zsq_ref[...] - 2.0 * dot + esq_ref[...]
    m = jnp.min(dists, axis=1)
    s = jnp.sum(m)
    oh = dists == m[:, None]
    iota_k = lax.broadcasted_iota(jnp.int32, (TILE, NUM_CODES), 1)
    idx_ref[0, 0, :] = jnp.min(jnp.where(oh, iota_k, NUM_CODES), axis=1)
    ohb = oh.astype(jnp.bfloat16)
    ones_r = jnp.ones((CNT_ROWS, TILE), jnp.bfloat16)
    oh8 = lax.dot_general(ones_r, ohb, (((1,), (0,)), ((), ())),
                          preferred_element_type=jnp.float32)

    @pl.when(i == 0)
    def _init():
        losssum_ref[0, 0] = s
        cnt_ref[...] = oh8

    @pl.when(i != 0)
    def _acc():
        losssum_ref[0, 0] += s
        cnt_ref[...] += oh8


_assign = pl.pallas_call(
    _assign_body,
    grid=(N_TILES,),
    in_specs=[
        pl.BlockSpec((TILE, CODE_DIM), lambda i: (i, 0)),
        pl.BlockSpec((TILE, 1), lambda i: (i, 0)),
        pl.BlockSpec((NUM_CODES, CODE_DIM), lambda i: (0, 0)),
        pl.BlockSpec((1, NUM_CODES), lambda i: (0, 0)),
    ],
    out_specs=[
        pl.BlockSpec((1, 1, TILE), lambda i: (i, 0, 0)),
        pl.BlockSpec((1, 1), lambda i: (0, 0), memory_space=pltpu.SMEM),
        pl.BlockSpec((CNT_ROWS, NUM_CODES), lambda i: (0, 0)),
    ],
    out_shape=[
        jax.ShapeDtypeStruct((N_TILES, 1, TILE), jnp.int32),
        jax.ShapeDtypeStruct((1, 1), jnp.float32),
        jax.ShapeDtypeStruct((CNT_ROWS, NUM_CODES), jnp.float32),
    ],
)


@functools.cache
def _make_sc_gather():
    return functools.partial(
        pl.kernel,
        mesh=plsc.VectorSubcoreMesh(core_axis_name="c", subcore_axis_name="s"),
        out_type=jax.ShapeDtypeStruct((N_TOK, CODE_DIM), jnp.float32),
        scratch_types=[
            pltpu.VMEM((CHUNKS_PER_W, CHUNK), jnp.int32),
            pltpu.VMEM((CHUNK, CODE_DIM), jnp.float32),
            pltpu.VMEM((CHUNK, CODE_DIM), jnp.float32),
            pltpu.SemaphoreType.DMA,
            pltpu.SemaphoreType.DMA,
        ],
    )(_sc_gather_body)


def _sc_gather_body(emb_hbm, idx_hbm, zq_hbm, idx_v, rows_a, rows_b, sem_a,
                    sem_b):
    c = lax.axis_index("c")
    s = lax.axis_index("s")
    wid = s * SC_NC + c
    pltpu.sync_copy(idx_hbm.at[wid], idx_v)

    bufs = (rows_a, rows_b)
    sems = (sem_a, sem_b)
    pltpu.async_copy(emb_hbm.at[idx_v.at[0]], rows_a, sem_a)
    for t in range(CHUNKS_PER_W):
        if t + 1 < CHUNKS_PER_W:
            pltpu.async_copy(emb_hbm.at[idx_v.at[t + 1]],
                             bufs[(t + 1) % 2], sems[(t + 1) % 2])
        pltpu.make_async_copy(emb_hbm.at[idx_v.at[t]], bufs[t % 2],
                              sems[t % 2]).wait()
        pltpu.sync_copy(bufs[t % 2],
                        zq_hbm.at[pl.ds(wid * TOK_PER_W + t * CHUNK, CHUNK)])


def _final_body(losssum_ref, cnt_ref, vq_ref, com_ref, cb_ref, perp_ref,
                use_ref):
    counts = jnp.sum(cnt_ref[...], axis=0)
    probs = jnp.maximum(counts / float(N_TOK), EPS)
    entropy = -jnp.sum(probs * jnp.log(probs))
    perp_ref[0, 0] = jnp.exp(entropy)
    use_ref[0, 0] = jnp.sum((counts > 0).astype(jnp.float32)) * (1.0 / NUM_CODES)
    ms = losssum_ref[0, 0] / float(N_TOK * CODE_DIM)
    com = jnp.minimum(BETA * ms, 100.0)
    cb = jnp.minimum(ms, 100.0)
    com_ref[0, 0] = com
    cb_ref[0, 0] = cb
    vq_ref[0, 0] = com + cb


_final = pl.pallas_call(
    _final_body,
    in_specs=[
        pl.BlockSpec((1, 1), lambda: (0, 0), memory_space=pltpu.SMEM),
        pl.BlockSpec((CNT_ROWS, NUM_CODES), lambda: (0, 0)),
    ],
    out_specs=[pl.BlockSpec((1, 1), lambda: (0, 0),
                            memory_space=pltpu.SMEM)] * 5,
    out_shape=[jax.ShapeDtypeStruct((1, 1), jnp.float32)] * 5,
)


def kernel(z_e, embedding):
    B, D, H, W = z_e.shape
    z = jnp.transpose(z_e, (0, 2, 3, 1))
    z_flat = z.reshape(-1, D)
    z_sq = jnp.sum(z_flat ** 2, axis=1, keepdims=True)
    e_sq = jnp.sum(embedding ** 2, axis=1)[None, :]
    idx3, losssum, cnt = _assign(z_flat, z_sq, embedding, e_sq)
    idx_flat = idx3.reshape(-1)
    zq_flat = _make_sc_gather()(
        embedding, idx_flat.reshape(SC_NW, CHUNKS_PER_W, CHUNK))
    vq, com, cb, perp, use = _final(losssum, cnt)
    z_q = jnp.transpose(zq_flat.reshape(B, H, W, D), (0, 3, 1, 2))
    idx_bt = idx_flat.reshape(B, H * W)[:, None, :]
    return (z_q, vq[0, 0], com[0, 0], cb[0, 0], perp[0, 0], use[0, 0], idx_bt)


# speedup vs baseline: 1.3329x; 1.0533x over previous
import functools

import jax
import jax.numpy as jnp
from jax import lax
from jax.experimental import pallas as pl
from jax.experimental.pallas import tpu as pltpu
from jax.experimental.pallas import tpu_sc as plsc

NUM_CODES = 8192
CODE_DIM = 256
N_TOK = 32 * 24 * 24
TILE = 1024
N_TILES = N_TOK // TILE
CNT_ROWS = 1
BETA = 0.25
EPS = 1e-05

SC_NC = 2
SC_NS = 16
SC_NW = SC_NC * SC_NS
TOK_PER_W = N_TOK // SC_NW
CHUNK = 96
CHUNKS_PER_W = TOK_PER_W // CHUNK


def _assign_body(z_ref, zsq_ref, emb_ref, esq_ref, idx_ref, losssum_ref,
                 cnt_ref):
    i = pl.program_id(0)
    z = z_ref[...]
    dot = lax.dot_general(z, emb_ref[...], (((1,), (1,)), ((), ())),
                          preferred_element_type=jnp.float32)
    dists = zsq_ref[...] - 2.0 * dot + esq_ref[...]
    m = jnp.min(dists, axis=1)
    s = jnp.sum(m)
    oh = dists == m[:, None]
    iota_k = lax.broadcasted_iota(jnp.int32, (TILE, NUM_CODES), 1)
    idx_ref[0, 0, :] = jnp.min(jnp.where(oh, iota_k, NUM_CODES), axis=1)
    ohb = oh.astype(jnp.bfloat16)
    ones_r = jnp.ones((CNT_ROWS, TILE), jnp.bfloat16)
    oh8 = lax.dot_general(ones_r, ohb, (((1,), (0,)), ((), ())),
                          preferred_element_type=jnp.float32)

    @pl.when(i == 0)
    def _init():
        losssum_ref[0, 0] = s
        cnt_ref[...] = oh8

    @pl.when(i != 0)
    def _acc():
        losssum_ref[0, 0] += s
        cnt_ref[...] += oh8


_assign = pl.pallas_call(
    _assign_body,
    grid=(N_TILES,),
    in_specs=[
        pl.BlockSpec((TILE, CODE_DIM), lambda i: (i, 0)),
        pl.BlockSpec((TILE, 1), lambda i: (i, 0)),
        pl.BlockSpec((NUM_CODES, CODE_DIM), lambda i: (0, 0)),
        pl.BlockSpec((1, NUM_CODES), lambda i: (0, 0)),
    ],
    out_specs=[
        pl.BlockSpec((1, 1, TILE), lambda i: (i, 0, 0)),
        pl.BlockSpec((1, 1), lambda i: (0, 0), memory_space=pltpu.SMEM),
        pl.BlockSpec((CNT_ROWS, NUM_CODES), lambda i: (0, 0)),
    ],
    out_shape=[
        jax.ShapeDtypeStruct((N_TILES, 1, TILE), jnp.int32),
        jax.ShapeDtypeStruct((1, 1), jnp.float32),
        jax.ShapeDtypeStruct((CNT_ROWS, NUM_CODES), jnp.float32),
    ],
)


@functools.cache
def _make_sc_gather():
    return functools.partial(
        pl.kernel,
        mesh=plsc.VectorSubcoreMesh(core_axis_name="c", subcore_axis_name="s"),
        out_type=jax.ShapeDtypeStruct((N_TOK, CODE_DIM), jnp.float32),
        scratch_types=[
            pltpu.VMEM((CHUNKS_PER_W, CHUNK), jnp.int32),
            pltpu.VMEM((CHUNK, CODE_DIM), jnp.float32),
            pltpu.VMEM((CHUNK, CODE_DIM), jnp.float32),
            pltpu.SemaphoreType.DMA,
            pltpu.SemaphoreType.DMA,
        ],
    )(_sc_gather_body)


def _sc_gather_body(emb_hbm, idx_hbm, zq_hbm, idx_v, rows_a, rows_b, sem_a,
                    sem_b):
    c = lax.axis_index("c")
    s = lax.axis_index("s")
    wid = s * SC_NC + c
    pltpu.sync_copy(idx_hbm.at[wid], idx_v)

    bufs = (rows_a, rows_b)
    sems = (sem_a, sem_b)
    pltpu.async_copy(emb_hbm.at[idx_v.at[0]], rows_a, sem_a)
    for t in range(CHUNKS_PER_W):
        if t + 1 < CHUNKS_PER_W:
            pltpu.async_copy(emb_hbm.at[idx_v.at[t + 1]],
                             bufs[(t + 1) % 2], sems[(t + 1) % 2])
        pltpu.make_async_copy(emb_hbm.at[idx_v.at[t]], bufs[t % 2],
                              sems[t % 2]).wait()
        pltpu.sync_copy(bufs[t % 2],
                        zq_hbm.at[pl.ds(wid * TOK_PER_W + t * CHUNK, CHUNK)])


def _final_body(losssum_ref, cnt_ref, vq_ref, com_ref, cb_ref, perp_ref,
                use_ref):
    counts = jnp.sum(cnt_ref[...], axis=0)
    probs = jnp.maximum(counts / float(N_TOK), EPS)
    entropy = -jnp.sum(probs * jnp.log(probs))
    perp_ref[0, 0] = jnp.exp(entropy)
    use_ref[0, 0] = jnp.sum((counts > 0).astype(jnp.float32)) * (1.0 / NUM_CODES)
    ms = losssum_ref[0, 0] / float(N_TOK * CODE_DIM)
    com = jnp.minimum(BETA * ms, 100.0)
    cb = jnp.minimum(ms, 100.0)
    com_ref[0, 0] = com
    cb_ref[0, 0] = cb
    vq_ref[0, 0] = com + cb


_final = pl.pallas_call(
    _final_body,
    in_specs=[
        pl.BlockSpec((1, 1), lambda: (0, 0), memory_space=pltpu.SMEM),
        pl.BlockSpec((CNT_ROWS, NUM_CODES), lambda: (0, 0)),
    ],
    out_specs=[pl.BlockSpec((1, 1), lambda: (0, 0),
                            memory_space=pltpu.SMEM)] * 5,
    out_shape=[jax.ShapeDtypeStruct((1, 1), jnp.float32)] * 5,
)


def kernel(z_e, embedding):
    B, D, H, W = z_e.shape
    z = jnp.transpose(z_e, (0, 2, 3, 1))
    z_flat = z.reshape(-1, D)
    z_sq = jnp.sum(z_flat ** 2, axis=1, keepdims=True)
    e_sq = jnp.sum(embedding ** 2, axis=1)[None, :]
    idx3, losssum, cnt = _assign(z_flat, z_sq, embedding, e_sq)
    idx_flat = idx3.reshape(-1)
    zq_flat = _make_sc_gather()(
        embedding, idx_flat.reshape(SC_NW, CHUNKS_PER_W, CHUNK))
    vq, com, cb, perp, use = _final(losssum, cnt)
    z_q = jnp.transpose(zq_flat.reshape(B, H, W, D), (0, 3, 1, 2))
    idx_bt = idx_flat.reshape(B, H * W)[:, None, :]
    return (z_q, vq[0, 0], com[0, 0], cb[0, 0], perp[0, 0], use[0, 0], idx_bt)


# speedup vs baseline: 1.3552x; 1.0167x over previous
import functools

import jax
import jax.numpy as jnp
from jax import lax
from jax.experimental import pallas as pl
from jax.experimental.pallas import tpu as pltpu
from jax.experimental.pallas import tpu_sc as plsc

NUM_CODES = 8192
CODE_DIM = 256
N_TOK = 32 * 24 * 24
TILE = 1152
N_TILES = N_TOK // TILE
CNT_ROWS = 1
BETA = 0.25
EPS = 1e-05

SC_NC = 2
SC_NS = 16
SC_NW = SC_NC * SC_NS
TOK_PER_W = N_TOK // SC_NW
CHUNK = 96
CHUNKS_PER_W = TOK_PER_W // CHUNK


def _assign_body(z_ref, zsq_ref, emb_ref, esq_ref, idx_ref, losssum_ref,
                 cnt_ref):
    i = pl.program_id(0)
    z = z_ref[...]
    dot = lax.dot_general(z, emb_ref[...], (((1,), (1,)), ((), ())),
                          preferred_element_type=jnp.float32)
    dists = zsq_ref[...] - 2.0 * dot + esq_ref[...]
    m = jnp.min(dists, axis=1)
    s = jnp.sum(m)
    oh = dists == m[:, None]
    iota_k = lax.broadcasted_iota(jnp.int32, (TILE, NUM_CODES), 1)
    idx_ref[0, 0, :] = jnp.min(jnp.where(oh, iota_k, NUM_CODES), axis=1)
    ohb = oh.astype(jnp.bfloat16)
    ones_r = jnp.ones((CNT_ROWS, TILE), jnp.bfloat16)
    oh8 = lax.dot_general(ones_r, ohb, (((1,), (0,)), ((), ())),
                          preferred_element_type=jnp.float32)

    @pl.when(i == 0)
    def _init():
        losssum_ref[0, 0] = s
        cnt_ref[...] = oh8

    @pl.when(i != 0)
    def _acc():
        losssum_ref[0, 0] += s
        cnt_ref[...] += oh8


_assign = pl.pallas_call(
    _assign_body,
    grid=(N_TILES,),
    in_specs=[
        pl.BlockSpec((TILE, CODE_DIM), lambda i: (i, 0)),
        pl.BlockSpec((TILE, 1), lambda i: (i, 0)),
        pl.BlockSpec((NUM_CODES, CODE_DIM), lambda i: (0, 0)),
        pl.BlockSpec((1, NUM_CODES), lambda i: (0, 0)),
    ],
    out_specs=[
        pl.BlockSpec((1, 1, TILE), lambda i: (i, 0, 0)),
        pl.BlockSpec((1, 1), lambda i: (0, 0), memory_space=pltpu.SMEM),
        pl.BlockSpec((CNT_ROWS, NUM_CODES), lambda i: (0, 0)),
    ],
    out_shape=[
        jax.ShapeDtypeStruct((N_TILES, 1, TILE), jnp.int32),
        jax.ShapeDtypeStruct((1, 1), jnp.float32),
        jax.ShapeDtypeStruct((CNT_ROWS, NUM_CODES), jnp.float32),
    ],
)


@functools.cache
def _make_sc_gather():
    return functools.partial(
        pl.kernel,
        mesh=plsc.VectorSubcoreMesh(core_axis_name="c", subcore_axis_name="s"),
        out_type=jax.ShapeDtypeStruct((N_TOK, CODE_DIM), jnp.float32),
        scratch_types=[
            pltpu.VMEM((CHUNKS_PER_W, CHUNK), jnp.int32),
            pltpu.VMEM((CHUNK, CODE_DIM), jnp.float32),
            pltpu.VMEM((CHUNK, CODE_DIM), jnp.float32),
            pltpu.SemaphoreType.DMA,
            pltpu.SemaphoreType.DMA,
        ],
    )(_sc_gather_body)


def _sc_gather_body(emb_hbm, idx_hbm, zq_hbm, idx_v, rows_a, rows_b, sem_a,
                    sem_b):
    c = lax.axis_index("c")
    s = lax.axis_index("s")
    wid = s * SC_NC + c
    pltpu.sync_copy(idx_hbm.at[wid], idx_v)

    bufs = (rows_a, rows_b)
    sems = (sem_a, sem_b)
    pltpu.async_copy(emb_hbm.at[idx_v.at[0]], rows_a, sem_a)
    for t in range(CHUNKS_PER_W):
        if t + 1 < CHUNKS_PER_W:
            pltpu.async_copy(emb_hbm.at[idx_v.at[t + 1]],
                             bufs[(t + 1) % 2], sems[(t + 1) % 2])
        pltpu.make_async_copy(emb_hbm.at[idx_v.at[t]], bufs[t % 2],
                              sems[t % 2]).wait()
        pltpu.sync_copy(bufs[t % 2],
                        zq_hbm.at[pl.ds(wid * TOK_PER_W + t * CHUNK, CHUNK)])


def _final_body(losssum_ref, cnt_ref, vq_ref, com_ref, cb_ref, perp_ref,
                use_ref):
    counts = jnp.sum(cnt_ref[...], axis=0)
    probs = jnp.maximum(counts / float(N_TOK), EPS)
    entropy = -jnp.sum(probs * jnp.log(probs))
    perp_ref[0, 0] = jnp.exp(entropy)
    use_ref[0, 0] = jnp.sum((counts > 0).astype(jnp.float32)) * (1.0 / NUM_CODES)
    ms = losssum_ref[0, 0] / float(N_TOK * CODE_DIM)
    com = jnp.minimum(BETA * ms, 100.0)
    cb = jnp.minimum(ms, 100.0)
    com_ref[0, 0] = com
    cb_ref[0, 0] = cb
    vq_ref[0, 0] = com + cb


_final = pl.pallas_call(
    _final_body,
    in_specs=[
        pl.BlockSpec((1, 1), lambda: (0, 0), memory_space=pltpu.SMEM),
        pl.BlockSpec((CNT_ROWS, NUM_CODES), lambda: (0, 0)),
    ],
    out_specs=[pl.BlockSpec((1, 1), lambda: (0, 0),
                            memory_space=pltpu.SMEM)] * 5,
    out_shape=[jax.ShapeDtypeStruct((1, 1), jnp.float32)] * 5,
)


def kernel(z_e, embedding):
    B, D, H, W = z_e.shape
    z = jnp.transpose(z_e, (0, 2, 3, 1))
    z_flat = z.reshape(-1, D)
    z_sq = jnp.sum(z_flat ** 2, axis=1, keepdims=True)
    e_sq = jnp.sum(embedding ** 2, axis=1)[None, :]
    idx3, losssum, cnt = _assign(z_flat, z_sq, embedding, e_sq)
    idx_flat = idx3.reshape(-1)
    zq_flat = _make_sc_gather()(
        embedding, idx_flat.reshape(SC_NW, CHUNKS_PER_W, CHUNK))
    vq, com, cb, perp, use = _final(losssum, cnt)
    z_q = jnp.transpose(zq_flat.reshape(B, H, W, D), (0, 3, 1, 2))
    idx_bt = idx_flat.reshape(B, H * W)[:, None, :]
    return (z_q, vq[0, 0], com[0, 0], cb[0, 0], perp[0, 0], use[0, 0], idx_bt)
